# Initial kernel scaffold; baseline (speedup 1.0000x reference)
import jax
import jax.numpy as jnp
from jax.experimental import pallas as pl


def kernel(x, edge_index, edge_attr, batch, params):
    raise NotImplementedError("write your pallas kernel here")


# speedup vs baseline: 18.8322x; 18.8322x over previous
import jax
import jax.numpy as jnp
from jax.experimental import pallas as pl
from jax.experimental.pallas import tpu as pltpu

N = 256
E0 = 4096
NODE_DIM = 12
HID = 128
NUM_LAYERS = 3
MAX_HOPS = 6
IB = 8
ECH = 1024

_F32 = jnp.float32
_HIGH = jax.lax.Precision.HIGHEST


def _ln(v, g, b):
    mu = jnp.mean(v, axis=-1, keepdims=True)
    var = jnp.mean((v - mu) * (v - mu), axis=-1, keepdims=True)
    return (v - mu) * jax.lax.rsqrt(var + 1e-5) * g + b


def _dot(a, b):
    return jnp.dot(a, b, preferred_element_type=_F32, precision=_HIGH)


def _bdot(a, b):
    return jnp.dot(a.astype(jnp.bfloat16), b.astype(jnp.bfloat16),
                   preferred_element_type=_F32)


def _bf(w):
    return w.astype(jnp.bfloat16).astype(_F32)


def _fna_body(*args):
    srow, dcol = args[1], args[2]
    (xp, new1, neb1, new2, neb2, neg, nebb) = (args[0],) + tuple(
        r[...] for r in args[3:9])
    xp = xp[...]
    layers = [tuple(r[...] for r in args[9 + 13 * l: 9 + 13 * (l + 1)])
              for l in range(NUM_LAYERS)]
    (dw1, db1, dw2, db2, lw1, lb1, lw2, lb2, dsc) = (
        r[...] for r in args[48:57])
    uo_ref, s_ref, ls_ref, sf_ref = args[57:61]
    hop_s, wm_s, brow_s = args[61:64]

    adj = jnp.zeros((N, N), _F32)
    for c in range(E0 // ECH):
        sr = srow[:, pl.ds(c * ECH, ECH)]
        dc = dcol[pl.ds(c * ECH, ECH), :]
        io_n = jax.lax.broadcasted_iota(jnp.int32, (N, ECH), 0)
        io_e = jax.lax.broadcasted_iota(jnp.int32, (ECH, N), 1)
        oh_s = (io_n == sr).astype(_F32)
        oh_d = (io_e == dc).astype(_F32)
        adj = adj + _bdot(oh_s, oh_d)
    adjb = (adj > 0.0).astype(_F32)

    r_i = jax.lax.broadcasted_iota(jnp.int32, (N, N), 0)
    c_i = jax.lax.broadcasted_iota(jnp.int32, (N, N), 1)
    hop = jnp.where(r_i == c_i, 0.0, 999.0).astype(_F32)
    hop = jnp.where(adjb > 0.0, 1.0, hop)
    curr = adjb
    for h in range(2, MAX_HOPS + 1):
        curr = (_bdot(curr, adjb) > 0.0).astype(_F32)
        hop = jnp.where((curr > 0.0) & (hop == 999.0), float(h), hop)
    wm = jnp.where(hop < 990.0, 1.0 / (hop * hop + 0.5), 0.0)
    hop_s[...] = hop
    wm_s[...] = wm
    ones_col = jnp.ones((N, 1), _F32)
    cnt = jax.lax.dot_general(wm, ones_col, (((0,), (0,)), ((), ())),
                              preferred_element_type=_F32,
                              precision=_HIGH)

    h1 = jnp.maximum(_bdot(xp, new1) + neb1, 0.0)
    xh = _ln(_bdot(h1, new2) + neb2, neg, nebb)

    for (m1a, m1b, wh, m1bias, m2w, m2b,
         u1a, u1bw, u1bias, u2w, u2bias, lng, lnb) in layers:
        acol = _bdot(xh, m1a) + m1bias
        brow_s[...] = _bdot(xh, m1b)
        whb = _bf(wh)

        def iblk(k, S):
            a0 = k * IB
            br = brow_s[pl.ds(a0, IB), :]
            hp = hop_s[pl.ds(a0, IB), :]
            wmb = wm_s[pl.ds(a0, IB), :]
            t = acol[None, :, :] + br[:, None, :] + hp[:, :, None] * whb[None, :, :]
            t = t * jax.nn.sigmoid(t)
            return S + jnp.sum(t * wmb[:, :, None], axis=0)

        S = jax.lax.fori_loop(0, N // IB, iblk, jnp.zeros((N, HID), _F32))
        agg = _dot(S, _bf(m2w)) + cnt * m2b
        pre = _bdot(xh, u1a) + _bdot(agg, u1bw) + u1bias
        u = _bdot(pre * jax.nn.sigmoid(pre), u2w) + u2bias
        xh = _ln(2.0 * xh + u, lng, lnb)

    d1 = jnp.maximum(_bdot(xh, dw1) + db1, 0.0)
    uo_ref[...] = (_bdot(d1, dw2) + db2) * dsc
    l1v = jnp.maximum(_bdot(xh, lw1) + lb1, 0.0)
    lsv = jnp.clip(_bdot(l1v, lw2) + lb2, 0.0, 30.0)
    sv = jnp.exp(lsv)
    ls_ref[...] = lsv
    s_ref[...] = sv
    sf_ref[...] = 250000000.0 / (sv + 1e-8)


def kernel(x, edge_index, edge_attr, batch, params):
    p = params
    xp = jnp.zeros((N, HID), _F32).at[:, :NODE_DIM].set(x)
    src_row = edge_index[0:1, :]
    dst_col = edge_index[1, :, None]

    ne = p["node_enc"]
    new1 = jnp.zeros((HID, HID), _F32).at[:NODE_DIM, :].set(ne["l1"]["w"])
    ins = [xp, src_row, dst_col,
           new1, ne["l1"]["b"][None, :],
           ne["l2"]["w"], ne["l2"]["b"][None, :],
           ne["ln"]["g"][None, :], ne["ln"]["b"][None, :]]
    for lp in p["layers"]:
        w1 = lp["msg1"]["w"]
        ins += [w1[0:HID, :], w1[HID:2 * HID, :], w1[3 * HID:3 * HID + 1, :],
                lp["msg1"]["b"][None, :],
                lp["msg2"]["w"], lp["msg2"]["b"][None, :],
                lp["upd1"]["w"][:HID, :], lp["upd1"]["w"][HID:, :],
                lp["upd1"]["b"][None, :],
                lp["upd2"]["w"], lp["upd2"]["b"][None, :],
                lp["ln"]["g"][None, :], lp["ln"]["b"][None, :]]
    dw1 = jnp.zeros((HID, HID), _F32).at[:, :HID // 2].set(p["disp1"]["w"])
    db1 = jnp.zeros((1, HID), _F32).at[0, :HID // 2].set(p["disp1"]["b"])
    dw2 = jnp.zeros((HID, HID), _F32).at[:HID // 2, :3].set(p["disp2"]["w"])
    db2 = jnp.zeros((1, HID), _F32).at[0, :3].set(p["disp2"]["b"])
    lw1 = jnp.zeros((HID, HID), _F32).at[:, :HID // 2].set(p["ls1"]["w"])
    lb1 = jnp.zeros((1, HID), _F32).at[0, :HID // 2].set(p["ls1"]["b"])
    lw2 = jnp.zeros((HID, HID), _F32).at[:HID // 2, :1].set(p["ls2"]["w"])
    lb2 = jnp.zeros((1, HID), _F32).at[0, :1].set(p["ls2"]["b"])
    disp_scale = 0.001 + jax.nn.softplus(p["log_disp_scale"])
    ins += [dw1, db1, dw2, db2, lw1, lb1, lw2, lb2,
            disp_scale.reshape(1, 1)]

    outs = pl.pallas_call(
        _fna_body,
        out_shape=[jax.ShapeDtypeStruct((N, HID), _F32) for _ in range(4)],
        scratch_shapes=[pltpu.VMEM((N, N), _F32),
                        pltpu.VMEM((N, N), _F32),
                        pltpu.VMEM((N, HID), _F32)],
    )(*ins)
    uo, sv, lsv, sfv = outs
    return (uo[:, :3], sv[:, :1], lsv[:, :1], disp_scale, sfv[:, :1])


# speedup vs baseline: 19.9041x; 1.0569x over previous
import jax
import jax.numpy as jnp
from jax.experimental import pallas as pl
from jax.experimental.pallas import tpu as pltpu

N = 256
E0 = 4096
NODE_DIM = 12
HID = 128
NUM_LAYERS = 3
MAX_HOPS = 6
IB = 16
ECH = 1024

_F32 = jnp.float32
_HIGH = jax.lax.Precision.HIGHEST


def _ln(v, g, b):
    mu = jnp.mean(v, axis=-1, keepdims=True)
    var = jnp.mean((v - mu) * (v - mu), axis=-1, keepdims=True)
    return (v - mu) * jax.lax.rsqrt(var + 1e-5) * g + b


def _dot(a, b):
    return jnp.dot(a, b, preferred_element_type=_F32, precision=_HIGH)


def _bdot(a, b):
    return jnp.dot(a.astype(jnp.bfloat16), b.astype(jnp.bfloat16),
                   preferred_element_type=_F32)


def _bf(w):
    return w.astype(jnp.bfloat16).astype(_F32)


def _fna_body(*args):
    srow, dcol = args[1], args[2]
    (xp, new1, neb1, new2, neb2, neg, nebb) = (args[0],) + tuple(
        r[...] for r in args[3:9])
    xp = xp[...]
    layers = [tuple(r[...] for r in args[9 + 13 * l: 9 + 13 * (l + 1)])
              for l in range(NUM_LAYERS)]
    (dw1, db1, dw2, db2, lw1, lb1, lw2, lb2, dsc) = (
        r[...] for r in args[48:57])
    uo_ref, s_ref, ls_ref, sf_ref = args[57:61]
    hop_s, wm_s, brow_s = args[61:64]

    adj = jnp.zeros((N, N), _F32)
    for c in range(E0 // ECH):
        sr = srow[:, pl.ds(c * ECH, ECH)]
        dc = dcol[pl.ds(c * ECH, ECH), :]
        io_n = jax.lax.broadcasted_iota(jnp.int32, (N, ECH), 0)
        io_e = jax.lax.broadcasted_iota(jnp.int32, (ECH, N), 1)
        oh_s = (io_n == sr).astype(_F32)
        oh_d = (io_e == dc).astype(_F32)
        adj = adj + _bdot(oh_s, oh_d)
    adjb = (adj > 0.0).astype(_F32)

    r_i = jax.lax.broadcasted_iota(jnp.int32, (N, N), 0)
    c_i = jax.lax.broadcasted_iota(jnp.int32, (N, N), 1)
    hop = jnp.where(r_i == c_i, 0.0, 999.0).astype(_F32)
    hop = jnp.where(adjb > 0.0, 1.0, hop)
    curr = adjb
    for h in range(2, MAX_HOPS + 1):
        curr = (_bdot(curr, adjb) > 0.0).astype(_F32)
        hop = jnp.where((curr > 0.0) & (hop == 999.0), float(h), hop)
    wm = jnp.where(hop < 990.0, 1.0 / (hop * hop + 0.5), 0.0)
    hop_s[...] = hop
    wm_s[...] = wm
    ones_col = jnp.ones((N, 1), _F32)
    cnt = jax.lax.dot_general(wm, ones_col, (((0,), (0,)), ((), ())),
                              preferred_element_type=_F32,
                              precision=_HIGH)

    h1 = jnp.maximum(_bdot(xp, new1) + neb1, 0.0)
    xh = _ln(_bdot(h1, new2) + neb2, neg, nebb)

    for (m1a, m1b, wh, m1bias, m2w, m2b,
         u1a, u1bw, u1bias, u2w, u2bias, lng, lnb) in layers:
        acol = _bdot(xh, m1a) + m1bias
        brow_s[...] = _bdot(xh, m1b)
        whb = _bf(wh)

        def iblk(k, S):
            a0 = k * IB
            br = brow_s[pl.ds(a0, IB), :]
            hp = hop_s[pl.ds(a0, IB), :]
            wmb = wm_s[pl.ds(a0, IB), :]
            t = acol[None, :, :] + br[:, None, :] + hp[:, :, None] * whb[None, :, :]
            t = t * jax.nn.sigmoid(t)
            return S + jnp.sum(t * wmb[:, :, None], axis=0)

        S = jax.lax.fori_loop(0, N // IB, iblk, jnp.zeros((N, HID), _F32), unroll=2)
        agg = _dot(S, _bf(m2w)) + cnt * m2b
        pre = _bdot(xh, u1a) + _bdot(agg, u1bw) + u1bias
        u = _bdot(pre * jax.nn.sigmoid(pre), u2w) + u2bias
        xh = _ln(2.0 * xh + u, lng, lnb)

    d1 = jnp.maximum(_bdot(xh, dw1) + db1, 0.0)
    uo_ref[...] = (_bdot(d1, dw2) + db2) * dsc
    l1v = jnp.maximum(_bdot(xh, lw1) + lb1, 0.0)
    lsv = jnp.clip(_bdot(l1v, lw2) + lb2, 0.0, 30.0)
    sv = jnp.exp(lsv)
    ls_ref[...] = lsv
    s_ref[...] = sv
    sf_ref[...] = 250000000.0 / (sv + 1e-8)


def kernel(x, edge_index, edge_attr, batch, params):
    p = params
    xp = jnp.zeros((N, HID), _F32).at[:, :NODE_DIM].set(x)
    src_row = edge_index[0:1, :]
    dst_col = edge_index[1, :, None]

    ne = p["node_enc"]
    new1 = jnp.zeros((HID, HID), _F32).at[:NODE_DIM, :].set(ne["l1"]["w"])
    ins = [xp, src_row, dst_col,
           new1, ne["l1"]["b"][None, :],
           ne["l2"]["w"], ne["l2"]["b"][None, :],
           ne["ln"]["g"][None, :], ne["ln"]["b"][None, :]]
    for lp in p["layers"]:
        w1 = lp["msg1"]["w"]
        ins += [w1[0:HID, :], w1[HID:2 * HID, :], w1[3 * HID:3 * HID + 1, :],
                lp["msg1"]["b"][None, :],
                lp["msg2"]["w"], lp["msg2"]["b"][None, :],
                lp["upd1"]["w"][:HID, :], lp["upd1"]["w"][HID:, :],
                lp["upd1"]["b"][None, :],
                lp["upd2"]["w"], lp["upd2"]["b"][None, :],
                lp["ln"]["g"][None, :], lp["ln"]["b"][None, :]]
    dw1 = jnp.zeros((HID, HID), _F32).at[:, :HID // 2].set(p["disp1"]["w"])
    db1 = jnp.zeros((1, HID), _F32).at[0, :HID // 2].set(p["disp1"]["b"])
    dw2 = jnp.zeros((HID, HID), _F32).at[:HID // 2, :3].set(p["disp2"]["w"])
    db2 = jnp.zeros((1, HID), _F32).at[0, :3].set(p["disp2"]["b"])
    lw1 = jnp.zeros((HID, HID), _F32).at[:, :HID // 2].set(p["ls1"]["w"])
    lb1 = jnp.zeros((1, HID), _F32).at[0, :HID // 2].set(p["ls1"]["b"])
    lw2 = jnp.zeros((HID, HID), _F32).at[:HID // 2, :1].set(p["ls2"]["w"])
    lb2 = jnp.zeros((1, HID), _F32).at[0, :1].set(p["ls2"]["b"])
    disp_scale = 0.001 + jax.nn.softplus(p["log_disp_scale"])
    ins += [dw1, db1, dw2, db2, lw1, lb1, lw2, lb2,
            disp_scale.reshape(1, 1)]

    outs = pl.pallas_call(
        _fna_body,
        out_shape=[jax.ShapeDtypeStruct((N, HID), _F32) for _ in range(4)],
        scratch_shapes=[pltpu.VMEM((N, N), _F32),
                        pltpu.VMEM((N, N), _F32),
                        pltpu.VMEM((N, HID), _F32)],
    )(*ins)
    uo, sv, lsv, sfv = outs
    return (uo[:, :3], sv[:, :1], lsv[:, :1], disp_scale, sfv[:, :1])


# speedup vs baseline: 25.3122x; 1.2717x over previous
import jax
import jax.numpy as jnp
from jax.experimental import pallas as pl
from jax.experimental.pallas import tpu as pltpu

N = 256
E0 = 4096
NODE_DIM = 12
HID = 128
NUM_LAYERS = 3
MAX_HOPS = 6
IB = 16
ECH = 1024

_F32 = jnp.float32
_HIGH = jax.lax.Precision.HIGHEST


def _ln(v, g, b):
    mu = jnp.mean(v, axis=-1, keepdims=True)
    var = jnp.mean((v - mu) * (v - mu), axis=-1, keepdims=True)
    return (v - mu) * jax.lax.rsqrt(var + 1e-5) * g + b


def _dot(a, b):
    return jnp.dot(a, b, preferred_element_type=_F32, precision=_HIGH)


def _bdot(a, b):
    return jnp.dot(a.astype(jnp.bfloat16), b.astype(jnp.bfloat16),
                   preferred_element_type=_F32)


def _bf(w):
    return w.astype(jnp.bfloat16).astype(_F32)


def _fna_body(*args):
    srow, dcol = args[1], args[2]
    (xp, new1, neb1, new2, neb2, neg, nebb) = (args[0],) + tuple(
        r[...] for r in args[3:9])
    xp = xp[...]
    layers = [tuple(r[...] for r in args[9 + 13 * l: 9 + 13 * (l + 1)])
              for l in range(NUM_LAYERS)]
    (dw1, db1, dw2, db2, lw1, lb1, lw2, lb2, dsc) = (
        r[...] for r in args[48:57])
    uo_ref, s_ref, ls_ref, sf_ref = args[57:61]
    hop_s, wm_s, brow_s = args[61:64]

    adj = jnp.zeros((N, N), _F32)
    for c in range(E0 // ECH):
        sr = srow[:, pl.ds(c * ECH, ECH)]
        dc = dcol[pl.ds(c * ECH, ECH), :]
        io_n = jax.lax.broadcasted_iota(jnp.int32, (N, ECH), 0)
        io_e = jax.lax.broadcasted_iota(jnp.int32, (ECH, N), 1)
        oh_s = (io_n == sr).astype(_F32)
        oh_d = (io_e == dc).astype(_F32)
        adj = adj + _bdot(oh_s, oh_d)
    adjb = (adj > 0.0).astype(_F32)

    r_i = jax.lax.broadcasted_iota(jnp.int32, (N, N), 0)
    c_i = jax.lax.broadcasted_iota(jnp.int32, (N, N), 1)
    hop = jnp.where(r_i == c_i, 0.0, 999.0).astype(_F32)
    hop = jnp.where(adjb > 0.0, 1.0, hop)
    curr = adjb
    for h in range(2, MAX_HOPS + 1):
        curr = (_bdot(curr, adjb) > 0.0).astype(_F32)
        hop = jnp.where((curr > 0.0) & (hop == 999.0), float(h), hop)
    wm = jnp.where(hop < 990.0, 1.0 / (hop * hop + 0.5), 0.0)
    hop_s[...] = hop.astype(jnp.bfloat16)
    wm_s[...] = wm.astype(jnp.bfloat16)
    ones_col = jnp.ones((N, 1), _F32)
    cnt = jax.lax.dot_general(wm, ones_col, (((0,), (0,)), ((), ())),
                              preferred_element_type=_F32,
                              precision=_HIGH)

    h1 = jnp.maximum(_bdot(xp, new1) + neb1, 0.0)
    xh = _ln(_bdot(h1, new2) + neb2, neg, nebb)

    for (m1a, m1b, wh, m1bias, m2w, m2b,
         u1a, u1bw, u1bias, u2w, u2bias, lng, lnb) in layers:
        acol = (_bdot(xh, m1a) + m1bias).astype(jnp.bfloat16)
        brow_s[...] = _bdot(xh, m1b).astype(jnp.bfloat16)
        whb = wh.astype(jnp.bfloat16)

        def iblk(k, S):
            a0 = k * IB
            br = brow_s[pl.ds(a0, IB), :]
            hp = hop_s[pl.ds(a0, IB), :]
            wmb = wm_s[pl.ds(a0, IB), :]
            t = acol[None, :, :] + br[:, None, :] + hp[:, :, None] * whb[None, :, :]
            t = t * jax.nn.sigmoid(t) * wmb[:, :, None]
            return S + jnp.sum(t, axis=0, dtype=_F32)

        S = jax.lax.fori_loop(0, N // IB, iblk, jnp.zeros((N, HID), _F32), unroll=2)
        agg = _dot(S, _bf(m2w)) + cnt * m2b
        pre = _bdot(xh, u1a) + _bdot(agg, u1bw) + u1bias
        u = _bdot(pre * jax.nn.sigmoid(pre), u2w) + u2bias
        xh = _ln(2.0 * xh + u, lng, lnb)

    d1 = jnp.maximum(_bdot(xh, dw1) + db1, 0.0)
    uo_ref[...] = (_bdot(d1, dw2) + db2) * dsc
    l1v = jnp.maximum(_bdot(xh, lw1) + lb1, 0.0)
    lsv = jnp.clip(_bdot(l1v, lw2) + lb2, 0.0, 30.0)
    sv = jnp.exp(lsv)
    ls_ref[...] = lsv
    s_ref[...] = sv
    sf_ref[...] = 250000000.0 / (sv + 1e-8)


def kernel(x, edge_index, edge_attr, batch, params):
    p = params
    xp = jnp.zeros((N, HID), _F32).at[:, :NODE_DIM].set(x)
    src_row = edge_index[0:1, :]
    dst_col = edge_index[1, :, None]

    ne = p["node_enc"]
    new1 = jnp.zeros((HID, HID), _F32).at[:NODE_DIM, :].set(ne["l1"]["w"])
    ins = [xp, src_row, dst_col,
           new1, ne["l1"]["b"][None, :],
           ne["l2"]["w"], ne["l2"]["b"][None, :],
           ne["ln"]["g"][None, :], ne["ln"]["b"][None, :]]
    for lp in p["layers"]:
        w1 = lp["msg1"]["w"]
        ins += [w1[0:HID, :], w1[HID:2 * HID, :], w1[3 * HID:3 * HID + 1, :],
                lp["msg1"]["b"][None, :],
                lp["msg2"]["w"], lp["msg2"]["b"][None, :],
                lp["upd1"]["w"][:HID, :], lp["upd1"]["w"][HID:, :],
                lp["upd1"]["b"][None, :],
                lp["upd2"]["w"], lp["upd2"]["b"][None, :],
                lp["ln"]["g"][None, :], lp["ln"]["b"][None, :]]
    dw1 = jnp.zeros((HID, HID), _F32).at[:, :HID // 2].set(p["disp1"]["w"])
    db1 = jnp.zeros((1, HID), _F32).at[0, :HID // 2].set(p["disp1"]["b"])
    dw2 = jnp.zeros((HID, HID), _F32).at[:HID // 2, :3].set(p["disp2"]["w"])
    db2 = jnp.zeros((1, HID), _F32).at[0, :3].set(p["disp2"]["b"])
    lw1 = jnp.zeros((HID, HID), _F32).at[:, :HID // 2].set(p["ls1"]["w"])
    lb1 = jnp.zeros((1, HID), _F32).at[0, :HID // 2].set(p["ls1"]["b"])
    lw2 = jnp.zeros((HID, HID), _F32).at[:HID // 2, :1].set(p["ls2"]["w"])
    lb2 = jnp.zeros((1, HID), _F32).at[0, :1].set(p["ls2"]["b"])
    disp_scale = 0.001 + jax.nn.softplus(p["log_disp_scale"])
    ins += [dw1, db1, dw2, db2, lw1, lb1, lw2, lb2,
            disp_scale.reshape(1, 1)]

    outs = pl.pallas_call(
        _fna_body,
        out_shape=[jax.ShapeDtypeStruct((N, HID), _F32) for _ in range(4)],
        scratch_shapes=[pltpu.VMEM((N, N), jnp.bfloat16),
                        pltpu.VMEM((N, N), jnp.bfloat16),
                        pltpu.VMEM((N, HID), jnp.bfloat16)],
    )(*ins)
    uo, sv, lsv, sfv = outs
    return (uo[:, :3], sv[:, :1], lsv[:, :1], disp_scale, sfv[:, :1])
